# Initial kernel scaffold; baseline (speedup 1.0000x reference)
import jax
import jax.numpy as jnp
from jax.experimental import pallas as pl


def kernel(x, edge_index, batch, x_label_change, edge_index_label_change, x_label_change_batch, x_label_unchange, edge_index_label_unchange, x_label_unchange_batch, W1, b1, W2, b2, W3, b3):
    raise NotImplementedError("write your pallas kernel here")


# speedup vs baseline: 12.8182x; 12.8182x over previous
import functools

import jax
import jax.numpy as jnp
from jax import lax
from jax.experimental import pallas as pl
from jax.experimental.pallas import tpu as pltpu
from jax.experimental.pallas import tpu_sc as plsc

N = 10000
D = 128
H1 = 32
H2 = 64
G = 64
NA = 3 * N
NP = 30720
E = 320000
EA = 3 * E
NC = 2
NS = 16
NW = NC * NS
CK = 128
CH = 235
EP = NW * CH * CK
RPT = NP // NS
GT = 256
BLK = 128
NBLK = NP // BLK

_mesh = plsc.VectorSubcoreMesh(
    core_axis_name="c", subcore_axis_name="s", num_cores=NC, num_subcores=NS)
_sc_params = pltpu.CompilerParams(use_tc_tiling_on_sc=False)


def _degree(dst_all):
    @functools.partial(
        pl.kernel,
        out_type=jax.ShapeDtypeStruct((NC, NP), jnp.float32),
        mesh=_mesh,
        compiler_params=_sc_params,
        scratch_types=[
            pltpu.VMEM((CH, CK), jnp.int32),
            pltpu.VMEM((CK,), jnp.float32),
            pltpu.VMEM((CK,), jnp.float32),
            pltpu.VMEM_SHARED((NP,), jnp.float32),
        ],
    )
    def deg_k(dst_hbm, degp_hbm, idx_v, ones_v, zb_v, acc):
        cid = lax.axis_index("c")
        sid = lax.axis_index("s")
        wid = sid * NC + cid
        z16 = jnp.zeros((16,), jnp.float32)
        o16 = jnp.ones((16,), jnp.float32)
        for i in range(CK // 16):
            zb_v[pl.ds(i * 16, 16)] = z16
            ones_v[pl.ds(i * 16, 16)] = o16

        def zbody(k, c):
            pltpu.sync_copy(zb_v, acc.at[pl.ds(sid * RPT + k * CK, CK)])
            return c
        lax.fori_loop(0, RPT // CK, zbody, None)
        pltpu.sync_copy(dst_hbm.at[wid], idx_v)
        plsc.subcore_barrier()

        def ebody(j, c):
            pltpu.sync_copy(ones_v, acc.at[idx_v.at[j]], add=True)
            return c
        lax.fori_loop(0, CH, ebody, None)
        plsc.subcore_barrier()
        pltpu.sync_copy(acc.at[pl.ds(sid * RPT, RPT)],
                        degp_hbm.at[cid, pl.ds(sid * RPT, RPT)])

    return deg_k(dst_all)


F = 32


def _propagate(y, src_all, dst_all):
    @functools.partial(
        pl.kernel,
        out_type=jax.ShapeDtypeStruct((NC, NP, F), jnp.float32),
        mesh=_mesh,
        compiler_params=_sc_params,
        scratch_types=[
            pltpu.VMEM((CH, CK), jnp.int32),
            pltpu.VMEM((CH, CK), jnp.int32),
            pltpu.VMEM((CK, F), jnp.float32),
            pltpu.VMEM((16, F), jnp.float32),
            pltpu.VMEM_SHARED((NP, F), jnp.float32),
            pltpu.SemaphoreType.DMA,
        ],
    )
    def prop_k(y_hbm, src_hbm, dst_hbm, p_hbm, sidx, didx, rows, zb, acc, sem):
        cid = lax.axis_index("c")
        sid = lax.axis_index("s")
        wid = sid * NC + cid
        z16 = jnp.zeros((16,), jnp.float32)
        for r in range(16):
            for cc in range(F // 16):
                zb[r, pl.ds(cc * 16, 16)] = z16

        def zbody(k, c):
            pltpu.sync_copy(zb, acc.at[pl.ds(sid * RPT + k * 16, 16)])
            return c
        lax.fori_loop(0, RPT // 16, zbody, None)
        pltpu.sync_copy(src_hbm.at[wid], sidx)
        pltpu.sync_copy(dst_hbm.at[wid], didx)
        plsc.subcore_barrier()

        def ebody(j, c):
            pltpu.async_copy(y_hbm.at[sidx.at[j]], rows, sem).wait()
            pltpu.sync_copy(rows, acc.at[didx.at[j]], add=True)
            return c
        lax.fori_loop(0, CH, ebody, None)
        plsc.subcore_barrier()
        pltpu.sync_copy(acc.at[pl.ds(sid * RPT, RPT)],
                        p_hbm.at[cid, pl.ds(sid * RPT, RPT)])

    return prop_k(y, src_all, dst_all)


def _dinv_col(dv_row):
    ii = lax.broadcasted_iota(jnp.int32, (BLK, BLK), 0)
    jj = lax.broadcasted_iota(jnp.int32, (BLK, BLK), 1)
    ident = (ii == jj).astype(jnp.float32)
    return lax.dot_general(ident, dv_row, (((1,), (1,)), ((), ())),
                           preferred_element_type=jnp.float32)


def _prep(degp, x_all, W1):
    def body(deg_ref, x_ref, w_ref, y_ref, dinv_ref):
        dsum = deg_ref[0:1, :] + deg_ref[1:2, :] + 1.0
        dv = lax.rsqrt(dsum)
        dinv_ref[...] = dv
        xw = jnp.dot(x_ref[...], w_ref[...],
                     preferred_element_type=jnp.float32)
        y_ref[...] = _dinv_col(dv) * xw

    return pl.pallas_call(
        body,
        grid=(NBLK,),
        in_specs=[
            pl.BlockSpec((NC, BLK), lambda i: (0, i)),
            pl.BlockSpec((BLK, D), lambda i: (i, 0)),
            pl.BlockSpec((D, H1), lambda i: (0, 0)),
        ],
        out_specs=[
            pl.BlockSpec((BLK, H1), lambda i: (i, 0)),
            pl.BlockSpec((1, BLK), lambda i: (0, i)),
        ],
        out_shape=[
            jax.ShapeDtypeStruct((NP, H1), jnp.float32),
            jax.ShapeDtypeStruct((1, NP), jnp.float32),
        ],
    )(degp, x_all, W1)


def _combine(ps, ys, dinv2d, b, Wn, F2):
    nh = len(ps)
    no = F2 // F

    def body(*refs):
        p_refs = refs[:nh]
        y_refs = refs[nh:2 * nh]
        dv_ref, b_ref, w_ref = refs[2 * nh:2 * nh + 3]
        yn_refs = refs[2 * nh + 3:]
        s = jnp.concatenate(
            [p_refs[k][0] + p_refs[k][1] + y_refs[k][...] for k in range(nh)],
            axis=1)
        dcol = _dinv_col(dv_ref[...])
        h = jnp.maximum(dcol * s + b_ref[...], 0.0)
        xwn = jnp.dot(h, w_ref[...], preferred_element_type=jnp.float32)
        yn = dcol * xwn
        for k in range(no):
            yn_refs[k][...] = yn[:, k * F:(k + 1) * F]

    return pl.pallas_call(
        body,
        grid=(NBLK,),
        in_specs=(
            [pl.BlockSpec((NC, BLK, F), lambda i: (0, i, 0))] * nh
            + [pl.BlockSpec((BLK, F), lambda i: (i, 0))] * nh
            + [
                pl.BlockSpec((1, BLK), lambda i: (0, i)),
                pl.BlockSpec((1, nh * F), lambda i: (0, 0)),
                pl.BlockSpec((nh * F, F2), lambda i: (0, 0)),
            ]
        ),
        out_specs=[pl.BlockSpec((BLK, F), lambda i: (i, 0))] * no,
        out_shape=[jax.ShapeDtypeStruct((NP, F), jnp.float32)] * no,
    )(*ps, *ys, dinv2d, b, Wn)


def _pool(ps, ys, dinv2d, b, seg2d):
    def body(plo_ref, phi_ref, ylo_ref, yhi_ref, dv_ref, b_ref, seg_ref,
             out_ref, acc_s, acc_c):
        i = pl.program_id(0)
        s = jnp.concatenate(
            [plo_ref[0] + plo_ref[1] + ylo_ref[...],
             phi_ref[0] + phi_ref[1] + yhi_ref[...]], axis=1)
        dcol = _dinv_col(dv_ref[...])
        h = jnp.maximum(dcol * s + b_ref[...], 0.0)
        gi = lax.broadcasted_iota(jnp.int32, (GT, BLK), 0)
        oh = (gi == seg_ref[...]).astype(jnp.float32)
        ps = jnp.dot(oh, h, preferred_element_type=jnp.float32)
        pc = jnp.dot(oh, jnp.ones((BLK, H2), jnp.float32),
                     preferred_element_type=jnp.float32)

        @pl.when(i == 0)
        def _():
            acc_s[...] = ps
            acc_c[...] = pc

        @pl.when(i > 0)
        def _():
            acc_s[...] = acc_s[...] + ps
            acc_c[...] = acc_c[...] + pc

        @pl.when(i == NBLK - 1)
        def _():
            out_ref[...] = acc_s[...] / jnp.maximum(acc_c[...], 1.0)

    return pl.pallas_call(
        body,
        grid=(NBLK,),
        in_specs=[
            pl.BlockSpec((NC, BLK, F), lambda i: (0, i, 0)),
            pl.BlockSpec((NC, BLK, F), lambda i: (0, i, 0)),
            pl.BlockSpec((BLK, F), lambda i: (i, 0)),
            pl.BlockSpec((BLK, F), lambda i: (i, 0)),
            pl.BlockSpec((1, BLK), lambda i: (0, i)),
            pl.BlockSpec((1, H2), lambda i: (0, 0)),
            pl.BlockSpec((1, BLK), lambda i: (0, i)),
        ],
        out_specs=pl.BlockSpec((GT, H2), lambda i: (0, 0)),
        out_shape=jax.ShapeDtypeStruct((GT, H2), jnp.float32),
        scratch_shapes=[
            pltpu.VMEM((GT, H2), jnp.float32),
            pltpu.VMEM((GT, H2), jnp.float32),
        ],
    )(ps[0], ps[1], ys[0], ys[1], dinv2d, b, seg2d)


def kernel(x, edge_index, batch,
           x_label_change, edge_index_label_change, x_label_change_batch,
           x_label_unchange, edge_index_label_unchange, x_label_unchange_batch,
           W1, b1, W2, b2, W3, b3):
    i32 = jnp.int32
    f32 = jnp.float32

    x_all = jnp.concatenate(
        [x, x_label_change, x_label_unchange,
         jnp.zeros((NP - NA, D), f32)], axis=0)

    pad_idx = NA + (jnp.arange(EP - EA, dtype=i32) % 512)
    src_all = jnp.concatenate(
        [edge_index[0].astype(i32),
         edge_index_label_change[0].astype(i32) + N,
         edge_index_label_unchange[0].astype(i32) + 2 * N,
         pad_idx]).reshape(NW, CH, CK)
    dst_all = jnp.concatenate(
        [edge_index[1].astype(i32),
         edge_index_label_change[1].astype(i32) + N,
         edge_index_label_unchange[1].astype(i32) + 2 * N,
         pad_idx]).reshape(NW, CH, CK)

    seg2d = jnp.concatenate(
        [batch.astype(i32),
         x_label_change_batch.astype(i32) + G,
         x_label_unchange_batch.astype(i32) + 2 * G,
         jnp.full((NP - NA,), 3 * G, i32)]).reshape(1, NP)

    degp = _degree(dst_all)
    y1, dinv2d = _prep(degp, x_all, W1)
    p1 = _propagate(y1, src_all, dst_all)
    y2 = _combine([p1], [y1], dinv2d, b1.reshape(1, H1), W2, H2)
    p2 = [_propagate(yk, src_all, dst_all) for yk in y2]
    y3 = _combine(p2, y2, dinv2d, b2.reshape(1, H2), W3, H2)
    p3 = [_propagate(yk, src_all, dst_all) for yk in y3]
    pooled = _pool(p3, y3, dinv2d, b3.reshape(1, H2), seg2d)
    return (pooled[0:G], pooled[G:2 * G], pooled[2 * G:3 * G])
